# baseline (speedup 1.0000x reference)
import functools

import jax
import jax.numpy as jnp
from jax import lax
from jax.experimental import pallas as pl
from jax.experimental.pallas import tpu as pltpu
from jax.experimental.pallas import tpu_sc as plsc

NE = 16
NK = 2
DM = 1024
DF = 4096
NT = 2048
NP = NT * NK
BM = 256
NBLK = 32
MPAD = NBLK * BM
EPAD = 128
DFN = 1024
NJ = DF // DFN
NSTEP = NBLK * NJ

_HI = jax.lax.Precision.HIGHEST


def _router_body(x_ref, wg_ref, logits_ref, sel0_ref, sel1_ref, w0_ref, w1_ref):
    x = x_ref[...]
    logits = lax.dot_general(x, wg_ref[...], (((1,), (1,)), ((), ())),
                             preferred_element_type=jnp.float32)
    logits_ref[...] = logits
    lane = lax.broadcasted_iota(jnp.int32, (NT, EPAD), 1)
    valid = lane < NE
    neg = jnp.float32(-1e30)
    l = jnp.where(valid, logits, neg)
    big = jnp.int32(1 << 30)
    m1 = jnp.max(l, axis=1, keepdims=True)
    a1 = jnp.min(jnp.where(l == m1, lane, big), axis=1, keepdims=True)
    l2 = jnp.where(lane == a1, neg, l)
    m2 = jnp.max(l2, axis=1, keepdims=True)
    a2 = jnp.min(jnp.where(l2 == m2, lane, big), axis=1, keepdims=True)
    w0 = 1.0 / (1.0 + jnp.exp(m2 - m1))
    sel0_ref[...] = a1
    sel1_ref[...] = a2
    w0_ref[...] = w0
    w1_ref[...] = 1.0 - w0


def _router(x, wg_pad):
    return pl.pallas_call(
        _router_body,
        out_shape=(
            jax.ShapeDtypeStruct((NT, EPAD), jnp.float32),
            jax.ShapeDtypeStruct((NT, 1), jnp.int32),
            jax.ShapeDtypeStruct((NT, 1), jnp.int32),
            jax.ShapeDtypeStruct((NT, 1), jnp.float32),
            jax.ShapeDtypeStruct((NT, 1), jnp.float32),
        ),
    )(x, wg_pad)


def _prep_body(sel0_ref, sel1_ref, pos_ref, tbl_ref, nst_ref):
    e_iota = lax.broadcasted_iota(jnp.int32, (NE, NT), 0)
    oh0 = (sel0_ref[...] == e_iota).astype(jnp.float32)
    oh1 = (sel1_ref[...] == e_iota).astype(jnp.float32)

    C = 512
    r = lax.broadcasted_iota(jnp.int32, (C, C), 0)
    cc = lax.broadcasted_iota(jnp.int32, (C, C), 1)
    u = (r < cc).astype(jnp.float32)

    def cum_excl(oh):
        parts = []
        carry = jnp.zeros((NE, 1), jnp.float32)
        for i in range(NT // C):
            blk = oh[:, i * C:(i + 1) * C]
            parts.append(lax.dot_general(blk, u, (((1,), (0,)), ((), ())),
                                         preferred_element_type=jnp.float32,
                                         precision=_HI) + carry)
            carry = carry + jnp.sum(blk, axis=1, keepdims=True)
        return jnp.concatenate(parts, axis=1), carry

    rank0, cnt0 = cum_excl(oh0)
    rank1, cnt1 = cum_excl(oh1)
    rank1 = rank1 + cnt0
    cnt = cnt0 + cnt1

    cnt_i = cnt.astype(jnp.int32)
    nblk_i = (cnt_i + (BM - 1)) >> 8
    nblk = nblk_i.astype(jnp.float32)

    tri = lax.broadcasted_iota(jnp.int32, (NE, NE), 1)
    row = lax.broadcasted_iota(jnp.int32, (NE, NE), 0)
    l_strict = (tri < row).astype(jnp.float32)
    l_incl = (tri <= row).astype(jnp.float32)
    off_blk = lax.dot_general(l_strict, nblk, (((1,), (0,)), ((), ())),
                              preferred_element_type=jnp.float32, precision=_HI)
    cum_incl = lax.dot_general(l_incl, nblk, (((1,), (0,)), ((), ())),
                               preferred_element_type=jnp.float32, precision=_HI)
    off_slot = off_blk * float(BM)

    pos0 = jnp.sum(oh0 * (rank0 + off_slot), axis=0, keepdims=True)
    pos1 = jnp.sum(oh1 * (rank1 + off_slot), axis=0, keepdims=True)
    pos_ref[0:1, :] = pos0.astype(jnp.int32)
    pos_ref[1:2, :] = pos1.astype(jnp.int32)

    nact = jnp.sum(nblk_i, axis=(0, 1), keepdims=False).reshape(1, 1)
    nsteps = nact * NJ
    nst_ref[...] = nsteps

    s_i = lax.broadcasted_iota(jnp.int32, (1, NSTEP), 1)
    sf = jnp.minimum(s_i, nsteps - 1).astype(jnp.float32)
    steps_inc = cum_incl * float(NJ)
    steps_exc = off_blk * float(NJ)
    e_of_s = jnp.sum((steps_inc <= sf).astype(jnp.float32),
                     axis=0, keepdims=True)
    e16 = lax.broadcasted_iota(jnp.int32, (NE, NSTEP), 0)
    oh_es = (e16 == e_of_s.astype(jnp.int32)).astype(jnp.float32)
    nblk_s = jnp.sum(oh_es * nblk, axis=0, keepdims=True)
    sexc_s = jnp.sum(oh_es * steps_exc, axis=0, keepdims=True)
    boff_s = jnp.sum(oh_es * off_blk, axis=0, keepdims=True)
    local = sf - sexc_s
    j_s = jnp.zeros((1, NSTEP), jnp.float32)
    for k in range(1, NJ):
        j_s = j_s + (local >= float(k) * nblk_s).astype(jnp.float32)
    b_s = local - j_s * nblk_s
    tbl_ref[0:1, :] = (boff_s + b_s).astype(jnp.int32)
    tbl_ref[1:2, :] = j_s.astype(jnp.int32)
    tbl_ref[2:3, :] = e_of_s.astype(jnp.int32)
    tbl_ref[3:4, :] = b_s.astype(jnp.int32)
    tbl_ref[4:5, :] = (j_s == 0.0).astype(jnp.int32)
    tbl_ref[5:6, :] = (j_s == float(NJ - 1)).astype(jnp.int32)
    tbl_ref[6:7, :] = jnp.zeros((1, NSTEP), jnp.int32)
    tbl_ref[7:8, :] = jnp.zeros((1, NSTEP), jnp.int32)


def _prep(sel0_row, sel1_row):
    return pl.pallas_call(
        _prep_body,
        out_shape=(
            jax.ShapeDtypeStruct((NK, NT), jnp.int32),
            jax.ShapeDtypeStruct((8, NSTEP), jnp.int32),
            jax.ShapeDtypeStruct((1, 1), jnp.int32),
        ),
    )(sel0_row, sel1_row)


_TPW = NT // 32


def _sc_dispatch(x, pos):
    @functools.partial(
        pl.kernel,
        out_type=jax.ShapeDtypeStruct((MPAD, DM), jnp.float32),
        mesh=plsc.VectorSubcoreMesh(core_axis_name="c", subcore_axis_name="s"),
        scratch_types=[
            pltpu.VMEM((_TPW, DM), jnp.float32),
            pltpu.VMEM((_TPW,), jnp.int32),
            pltpu.SemaphoreType.DMA,
        ],
    )
    def body(x_hbm, pos_hbm, xs_hbm, xtile, idx, sem):
        wid = lax.axis_index("s") * 2 + lax.axis_index("c")
        base = wid * _TPW
        pltpu.sync_copy(x_hbm.at[pl.ds(base, _TPW)], xtile)
        pltpu.sync_copy(pos_hbm.at[0, pl.ds(base, _TPW)], idx)
        pltpu.async_copy(xtile, xs_hbm.at[idx], sem).wait()
        pltpu.sync_copy(pos_hbm.at[1, pl.ds(base, _TPW)], idx)
        pltpu.async_copy(xtile, xs_hbm.at[idx], sem).wait()

    return body(x, pos)


def _sc_gather(ys, pos):
    @functools.partial(
        pl.kernel,
        out_type=(
            jax.ShapeDtypeStruct((NT, DM), jnp.float32),
            jax.ShapeDtypeStruct((NT, DM), jnp.float32),
        ),
        mesh=plsc.VectorSubcoreMesh(core_axis_name="c", subcore_axis_name="s"),
        scratch_types=[
            pltpu.VMEM((_TPW, DM), jnp.float32),
            pltpu.VMEM((_TPW,), jnp.int32),
            pltpu.SemaphoreType.DMA,
        ],
    )
    def body(ys_hbm, pos_hbm, y0_hbm, y1_hbm, buf, idx, sem):
        wid = lax.axis_index("s") * 2 + lax.axis_index("c")
        base = wid * _TPW
        pltpu.sync_copy(pos_hbm.at[0, pl.ds(base, _TPW)], idx)
        pltpu.async_copy(ys_hbm.at[idx], buf, sem).wait()
        pltpu.sync_copy(buf, y0_hbm.at[pl.ds(base, _TPW)])
        pltpu.sync_copy(pos_hbm.at[1, pl.ds(base, _TPW)], idx)
        pltpu.async_copy(ys_hbm.at[idx], buf, sem).wait()
        pltpu.sync_copy(buf, y1_hbm.at[pl.ds(base, _TPW)])

    return body(ys, pos)


def _mlp_body(tbl_ref, nst_ref, xs_ref, wfc_ref, wproj_ref, out_ref, acc_ref):
    s = pl.program_id(0)

    @pl.when(s < nst_ref[0])
    def _():
        x = xs_ref[...]
        h = lax.dot_general(x, wfc_ref[0], (((1,), (1,)), ((), ())),
                            preferred_element_type=jnp.float32)
        h = jax.nn.gelu(h)
        y = lax.dot_general(h, wproj_ref[0], (((1,), (1,)), ((), ())),
                            preferred_element_type=jnp.float32)
        off = tbl_ref[3, s] * BM

        @pl.when(tbl_ref[4, s] == 1)
        def _():
            acc_ref[pl.ds(off, BM), :] = y

        @pl.when(tbl_ref[4, s] == 0)
        def _():
            acc_ref[pl.ds(off, BM), :] += y

        @pl.when(tbl_ref[5, s] == 1)
        def _():
            out_ref[...] = acc_ref[pl.ds(off, BM), :]


def _grouped_mlp(xs, wfc, wproj, tbl, nst):
    grid_spec = pltpu.PrefetchScalarGridSpec(
        num_scalar_prefetch=2,
        grid=(NSTEP,),
        in_specs=[
            pl.BlockSpec((BM, DM), lambda s, tbl, nst: (tbl[0, s], 0)),
            pl.BlockSpec((1, DFN, DM),
                         lambda s, tbl, nst: (tbl[2, s], tbl[1, s], 0)),
            pl.BlockSpec((1, DM, DFN),
                         lambda s, tbl, nst: (tbl[2, s], 0, tbl[1, s])),
        ],
        out_specs=pl.BlockSpec(
            (BM, DM),
            lambda s, tbl, nst: (jnp.where(tbl[5, s] == 1, tbl[0, s], NBLK), 0)),
        scratch_shapes=[pltpu.VMEM((NE * BM, DM), jnp.float32)],
    )
    full = pl.pallas_call(
        _mlp_body,
        grid_spec=grid_spec,
        out_shape=jax.ShapeDtypeStruct((MPAD + BM, DM), jnp.float32),
    )(tbl, nst, xs, wfc, wproj)
    return full


def _combine_body(y0_ref, y1_ref, w0_ref, w1_ref, out_ref):
    out_ref[...] = w0_ref[...] * y0_ref[...] + w1_ref[...] * y1_ref[...]


def _combine(y0, y1, w0, w1):
    nb = 8
    bt = NT // nb
    return pl.pallas_call(
        _combine_body,
        grid=(nb,),
        in_specs=[
            pl.BlockSpec((bt, DM), lambda i: (i, 0)),
            pl.BlockSpec((bt, DM), lambda i: (i, 0)),
            pl.BlockSpec((bt, 1), lambda i: (i, 0)),
            pl.BlockSpec((bt, 1), lambda i: (i, 0)),
        ],
        out_specs=pl.BlockSpec((bt, DM), lambda i: (i, 0)),
        out_shape=jax.ShapeDtypeStruct((NT, DM), jnp.float32),
    )(y0, y1, w0, w1)


def kernel(hidden_states, Wg, Wfc, Wproj):
    Bc, Sc, Dc = hidden_states.shape
    x = hidden_states.reshape(-1, Dc)

    wg_pad = jnp.zeros((EPAD, DM), jnp.float32).at[:NE].set(Wg)
    logits_pad, sel0, sel1, w0, w1 = _router(x, wg_pad)
    router_logits = logits_pad[:, :NE]

    pos, tbl, nst = _prep(sel0.reshape(1, NT), sel1.reshape(1, NT))

    xs = _sc_dispatch(x, pos)
    ys = _grouped_mlp(xs, Wfc, Wproj, tbl, nst.reshape(1))
    y0, y1 = _sc_gather(ys, pos)
    out = _combine(y0, y1, w0, w1)

    return out.reshape(Bc, Sc, Dc), router_logits


# speedup vs baseline: 1.1131x; 1.1131x over previous
import functools

import jax
import jax.numpy as jnp
from jax import lax
from jax.experimental import pallas as pl
from jax.experimental.pallas import tpu as pltpu
from jax.experimental.pallas import tpu_sc as plsc

NE = 16
NK = 2
DM = 1024
DF = 4096
NT = 2048
NP = NT * NK
BM = 256
NBLK = 32
MPAD = NBLK * BM
EPAD = 128
DFN = 2048
NJ = DF // DFN
NSTEP = NBLK * NJ

_HI = jax.lax.Precision.HIGHEST


def _router_body(x_ref, wg_ref, logits_ref, sel0_ref, sel1_ref, w0_ref, w1_ref):
    x = x_ref[...]
    logits = lax.dot_general(x, wg_ref[...], (((1,), (1,)), ((), ())),
                             preferred_element_type=jnp.float32)
    logits_ref[...] = logits
    lane = lax.broadcasted_iota(jnp.int32, (NT, EPAD), 1)
    valid = lane < NE
    neg = jnp.float32(-1e30)
    l = jnp.where(valid, logits, neg)
    big = jnp.int32(1 << 30)
    m1 = jnp.max(l, axis=1, keepdims=True)
    a1 = jnp.min(jnp.where(l == m1, lane, big), axis=1, keepdims=True)
    l2 = jnp.where(lane == a1, neg, l)
    m2 = jnp.max(l2, axis=1, keepdims=True)
    a2 = jnp.min(jnp.where(l2 == m2, lane, big), axis=1, keepdims=True)
    w0 = 1.0 / (1.0 + jnp.exp(m2 - m1))
    sel0_ref[...] = a1
    sel1_ref[...] = a2
    w0_ref[...] = w0
    w1_ref[...] = 1.0 - w0


def _router(x, wg_pad):
    return pl.pallas_call(
        _router_body,
        out_shape=(
            jax.ShapeDtypeStruct((NT, EPAD), jnp.float32),
            jax.ShapeDtypeStruct((NT, 1), jnp.int32),
            jax.ShapeDtypeStruct((NT, 1), jnp.int32),
            jax.ShapeDtypeStruct((NT, 1), jnp.float32),
            jax.ShapeDtypeStruct((NT, 1), jnp.float32),
        ),
    )(x, wg_pad)


def _prep_body(sel0_ref, sel1_ref, pos_ref, tbl_ref, nst_ref):
    e_iota = lax.broadcasted_iota(jnp.int32, (NE, NT), 0)
    oh0 = (sel0_ref[...] == e_iota).astype(jnp.float32)
    oh1 = (sel1_ref[...] == e_iota).astype(jnp.float32)

    C = 512
    r = lax.broadcasted_iota(jnp.int32, (C, C), 0)
    cc = lax.broadcasted_iota(jnp.int32, (C, C), 1)
    u = (r < cc).astype(jnp.float32)

    def cum_excl(oh):
        parts = []
        carry = jnp.zeros((NE, 1), jnp.float32)
        for i in range(NT // C):
            blk = oh[:, i * C:(i + 1) * C]
            parts.append(lax.dot_general(blk, u, (((1,), (0,)), ((), ())),
                                         preferred_element_type=jnp.float32,
                                         precision=_HI) + carry)
            carry = carry + jnp.sum(blk, axis=1, keepdims=True)
        return jnp.concatenate(parts, axis=1), carry

    rank0, cnt0 = cum_excl(oh0)
    rank1, cnt1 = cum_excl(oh1)
    rank1 = rank1 + cnt0
    cnt = cnt0 + cnt1

    cnt_i = cnt.astype(jnp.int32)
    nblk_i = (cnt_i + (BM - 1)) >> 8
    nblk = nblk_i.astype(jnp.float32)

    tri = lax.broadcasted_iota(jnp.int32, (NE, NE), 1)
    row = lax.broadcasted_iota(jnp.int32, (NE, NE), 0)
    l_strict = (tri < row).astype(jnp.float32)
    l_incl = (tri <= row).astype(jnp.float32)
    off_blk = lax.dot_general(l_strict, nblk, (((1,), (0,)), ((), ())),
                              preferred_element_type=jnp.float32, precision=_HI)
    cum_incl = lax.dot_general(l_incl, nblk, (((1,), (0,)), ((), ())),
                               preferred_element_type=jnp.float32, precision=_HI)
    off_slot = off_blk * float(BM)

    pos0 = jnp.sum(oh0 * (rank0 + off_slot), axis=0, keepdims=True)
    pos1 = jnp.sum(oh1 * (rank1 + off_slot), axis=0, keepdims=True)
    pos_ref[0:1, :] = pos0.astype(jnp.int32)
    pos_ref[1:2, :] = pos1.astype(jnp.int32)

    nact = jnp.sum(nblk_i, axis=(0, 1), keepdims=False).reshape(1, 1)
    nsteps = nact * NJ
    nst_ref[...] = nsteps

    s_i = lax.broadcasted_iota(jnp.int32, (1, NSTEP), 1)
    sf = jnp.minimum(s_i, nsteps - 1).astype(jnp.float32)
    steps_inc = cum_incl * float(NJ)
    steps_exc = off_blk * float(NJ)
    e_of_s = jnp.sum((steps_inc <= sf).astype(jnp.float32),
                     axis=0, keepdims=True)
    e16 = lax.broadcasted_iota(jnp.int32, (NE, NSTEP), 0)
    oh_es = (e16 == e_of_s.astype(jnp.int32)).astype(jnp.float32)
    nblk_s = jnp.sum(oh_es * nblk, axis=0, keepdims=True)
    sexc_s = jnp.sum(oh_es * steps_exc, axis=0, keepdims=True)
    boff_s = jnp.sum(oh_es * off_blk, axis=0, keepdims=True)
    local = sf - sexc_s
    j_s = jnp.zeros((1, NSTEP), jnp.float32)
    for k in range(1, NJ):
        j_s = j_s + (local >= float(k) * nblk_s).astype(jnp.float32)
    b_s = local - j_s * nblk_s
    tbl_ref[0:1, :] = (boff_s + b_s).astype(jnp.int32)
    tbl_ref[1:2, :] = j_s.astype(jnp.int32)
    tbl_ref[2:3, :] = e_of_s.astype(jnp.int32)
    tbl_ref[3:4, :] = b_s.astype(jnp.int32)
    tbl_ref[4:5, :] = (j_s == 0.0).astype(jnp.int32)
    tbl_ref[5:6, :] = (j_s == float(NJ - 1)).astype(jnp.int32)
    tbl_ref[6:7, :] = jnp.zeros((1, NSTEP), jnp.int32)
    tbl_ref[7:8, :] = jnp.zeros((1, NSTEP), jnp.int32)


def _prep(sel0_row, sel1_row):
    return pl.pallas_call(
        _prep_body,
        out_shape=(
            jax.ShapeDtypeStruct((NK, NT), jnp.int32),
            jax.ShapeDtypeStruct((8, NSTEP), jnp.int32),
            jax.ShapeDtypeStruct((1, 1), jnp.int32),
        ),
    )(sel0_row, sel1_row)


_TPW = NT // 32


def _sc_dispatch(x, pos):
    @functools.partial(
        pl.kernel,
        out_type=jax.ShapeDtypeStruct((MPAD, DM), jnp.float32),
        mesh=plsc.VectorSubcoreMesh(core_axis_name="c", subcore_axis_name="s"),
        scratch_types=[
            pltpu.VMEM((_TPW, DM), jnp.float32),
            pltpu.VMEM((_TPW,), jnp.int32),
            pltpu.SemaphoreType.DMA,
        ],
    )
    def body(x_hbm, pos_hbm, xs_hbm, xtile, idx, sem):
        wid = lax.axis_index("s") * 2 + lax.axis_index("c")
        base = wid * _TPW
        pltpu.sync_copy(x_hbm.at[pl.ds(base, _TPW)], xtile)
        pltpu.sync_copy(pos_hbm.at[0, pl.ds(base, _TPW)], idx)
        pltpu.async_copy(xtile, xs_hbm.at[idx], sem).wait()
        pltpu.sync_copy(pos_hbm.at[1, pl.ds(base, _TPW)], idx)
        pltpu.async_copy(xtile, xs_hbm.at[idx], sem).wait()

    return body(x, pos)


def _sc_gather(ys, pos):
    @functools.partial(
        pl.kernel,
        out_type=(
            jax.ShapeDtypeStruct((NT, DM), jnp.float32),
            jax.ShapeDtypeStruct((NT, DM), jnp.float32),
        ),
        mesh=plsc.VectorSubcoreMesh(core_axis_name="c", subcore_axis_name="s"),
        scratch_types=[
            pltpu.VMEM((_TPW, DM), jnp.float32),
            pltpu.VMEM((_TPW,), jnp.int32),
            pltpu.SemaphoreType.DMA,
        ],
    )
    def body(ys_hbm, pos_hbm, y0_hbm, y1_hbm, buf, idx, sem):
        wid = lax.axis_index("s") * 2 + lax.axis_index("c")
        base = wid * _TPW
        pltpu.sync_copy(pos_hbm.at[0, pl.ds(base, _TPW)], idx)
        pltpu.async_copy(ys_hbm.at[idx], buf, sem).wait()
        pltpu.sync_copy(buf, y0_hbm.at[pl.ds(base, _TPW)])
        pltpu.sync_copy(pos_hbm.at[1, pl.ds(base, _TPW)], idx)
        pltpu.async_copy(ys_hbm.at[idx], buf, sem).wait()
        pltpu.sync_copy(buf, y1_hbm.at[pl.ds(base, _TPW)])

    return body(ys, pos)


def _mlp_body(tbl_ref, nst_ref, xs_ref, wfc_ref, wproj_ref, out_ref, acc_ref):
    s = pl.program_id(0)

    @pl.when(s < nst_ref[0])
    def _():
        x = xs_ref[...]
        h = lax.dot_general(x, wfc_ref[0], (((1,), (1,)), ((), ())),
                            preferred_element_type=jnp.float32)
        h = jax.nn.gelu(h)
        y = lax.dot_general(h, wproj_ref[0], (((1,), (1,)), ((), ())),
                            preferred_element_type=jnp.float32)
        off = tbl_ref[3, s] * BM

        @pl.when(tbl_ref[4, s] == 1)
        def _():
            acc_ref[pl.ds(off, BM), :] = y

        @pl.when(tbl_ref[4, s] == 0)
        def _():
            acc_ref[pl.ds(off, BM), :] += y

        @pl.when(tbl_ref[5, s] == 1)
        def _():
            out_ref[...] = acc_ref[pl.ds(off, BM), :]


def _grouped_mlp(xs, wfc, wproj, tbl, nst):
    grid_spec = pltpu.PrefetchScalarGridSpec(
        num_scalar_prefetch=2,
        grid=(NSTEP,),
        in_specs=[
            pl.BlockSpec((BM, DM), lambda s, tbl, nst: (tbl[0, s], 0)),
            pl.BlockSpec((1, DFN, DM),
                         lambda s, tbl, nst: (tbl[2, s], tbl[1, s], 0)),
            pl.BlockSpec((1, DM, DFN),
                         lambda s, tbl, nst: (tbl[2, s], 0, tbl[1, s])),
        ],
        out_specs=pl.BlockSpec(
            (BM, DM),
            lambda s, tbl, nst: (jnp.where(tbl[5, s] == 1, tbl[0, s], NBLK), 0)),
        scratch_shapes=[pltpu.VMEM((NE * BM, DM), jnp.float32)],
    )
    full = pl.pallas_call(
        _mlp_body,
        grid_spec=grid_spec,
        out_shape=jax.ShapeDtypeStruct((MPAD + BM, DM), jnp.float32),
    )(tbl, nst, xs, wfc, wproj)
    return full


def _combine_body(y0_ref, y1_ref, w0_ref, w1_ref, out_ref):
    out_ref[...] = w0_ref[...] * y0_ref[...] + w1_ref[...] * y1_ref[...]


def _combine(y0, y1, w0, w1):
    nb = 8
    bt = NT // nb
    return pl.pallas_call(
        _combine_body,
        grid=(nb,),
        in_specs=[
            pl.BlockSpec((bt, DM), lambda i: (i, 0)),
            pl.BlockSpec((bt, DM), lambda i: (i, 0)),
            pl.BlockSpec((bt, 1), lambda i: (i, 0)),
            pl.BlockSpec((bt, 1), lambda i: (i, 0)),
        ],
        out_specs=pl.BlockSpec((bt, DM), lambda i: (i, 0)),
        out_shape=jax.ShapeDtypeStruct((NT, DM), jnp.float32),
    )(y0, y1, w0, w1)


def kernel(hidden_states, Wg, Wfc, Wproj):
    Bc, Sc, Dc = hidden_states.shape
    x = hidden_states.reshape(-1, Dc)

    wg_pad = jnp.zeros((EPAD, DM), jnp.float32).at[:NE].set(Wg)
    logits_pad, sel0, sel1, w0, w1 = _router(x, wg_pad)
    router_logits = logits_pad[:, :NE]

    pos, tbl, nst = _prep(sel0.reshape(1, NT), sel1.reshape(1, NT))

    xs = _sc_dispatch(x, pos)
    ys = _grouped_mlp(xs, Wfc, Wproj, tbl, nst.reshape(1))
    y0, y1 = _sc_gather(ys, pos)
    out = _combine(y0, y1, w0, w1)

    return out.reshape(Bc, Sc, Dc), router_logits


# speedup vs baseline: 1.1184x; 1.0047x over previous
import functools

import jax
import jax.numpy as jnp
from jax import lax
from jax.experimental import pallas as pl
from jax.experimental.pallas import tpu as pltpu
from jax.experimental.pallas import tpu_sc as plsc

NE = 16
NK = 2
DM = 1024
DF = 4096
NT = 2048
NP = NT * NK
BM = 256
NBLK = 32
MPAD = NBLK * BM
EPAD = 128
DFN = 2048
NJ = DF // DFN
NSTEP = NBLK * NJ

_HI = jax.lax.Precision.HIGHEST


def _router_body(x_ref, wg_ref, logits_ref, sel0_ref, sel1_ref, w0_ref, w1_ref):
    x = x_ref[...]
    logits = lax.dot_general(x, wg_ref[...], (((1,), (1,)), ((), ())),
                             preferred_element_type=jnp.float32)
    logits_ref[...] = logits
    lane = lax.broadcasted_iota(jnp.int32, (NT, NE), 1)
    neg = jnp.float32(-1e30)
    l = logits
    big = jnp.int32(1 << 30)
    m1 = jnp.max(l, axis=1, keepdims=True)
    a1 = jnp.min(jnp.where(l == m1, lane, big), axis=1, keepdims=True)
    l2 = jnp.where(lane == a1, neg, l)
    m2 = jnp.max(l2, axis=1, keepdims=True)
    a2 = jnp.min(jnp.where(l2 == m2, lane, big), axis=1, keepdims=True)
    w0 = 1.0 / (1.0 + jnp.exp(m2 - m1))
    sel0_ref[...] = a1
    sel1_ref[...] = a2
    w0_ref[...] = w0
    w1_ref[...] = 1.0 - w0


def _router(x, wg_pad):
    return pl.pallas_call(
        _router_body,
        out_shape=(
            jax.ShapeDtypeStruct((NT, NE), jnp.float32),
            jax.ShapeDtypeStruct((NT, 1), jnp.int32),
            jax.ShapeDtypeStruct((NT, 1), jnp.int32),
            jax.ShapeDtypeStruct((NT, 1), jnp.float32),
            jax.ShapeDtypeStruct((NT, 1), jnp.float32),
        ),
    )(x, wg_pad)


def _prep_body(sel0_ref, sel1_ref, pos_ref, tbl_ref, nst_ref):
    e_iota = lax.broadcasted_iota(jnp.int32, (NE, NT), 0)
    oh0 = (sel0_ref[...] == e_iota).astype(jnp.float32)
    oh1 = (sel1_ref[...] == e_iota).astype(jnp.float32)

    C = 512
    r = lax.broadcasted_iota(jnp.int32, (C, C), 0)
    cc = lax.broadcasted_iota(jnp.int32, (C, C), 1)
    u = (r < cc).astype(jnp.float32)

    def cum_excl(oh):
        parts = []
        carry = jnp.zeros((NE, 1), jnp.float32)
        for i in range(NT // C):
            blk = oh[:, i * C:(i + 1) * C]
            parts.append(lax.dot_general(blk, u, (((1,), (0,)), ((), ())),
                                         preferred_element_type=jnp.float32,
                                         precision=_HI) + carry)
            carry = carry + jnp.sum(blk, axis=1, keepdims=True)
        return jnp.concatenate(parts, axis=1), carry

    rank0, cnt0 = cum_excl(oh0)
    rank1, cnt1 = cum_excl(oh1)
    rank1 = rank1 + cnt0
    cnt = cnt0 + cnt1

    cnt_i = cnt.astype(jnp.int32)
    nblk_i = (cnt_i + (BM - 1)) >> 8
    nblk = nblk_i.astype(jnp.float32)

    tri = lax.broadcasted_iota(jnp.int32, (NE, NE), 1)
    row = lax.broadcasted_iota(jnp.int32, (NE, NE), 0)
    l_strict = (tri < row).astype(jnp.float32)
    l_incl = (tri <= row).astype(jnp.float32)
    off_blk = lax.dot_general(l_strict, nblk, (((1,), (0,)), ((), ())),
                              preferred_element_type=jnp.float32, precision=_HI)
    cum_incl = lax.dot_general(l_incl, nblk, (((1,), (0,)), ((), ())),
                               preferred_element_type=jnp.float32, precision=_HI)
    off_slot = off_blk * float(BM)

    pos0 = jnp.sum(oh0 * (rank0 + off_slot), axis=0, keepdims=True)
    pos1 = jnp.sum(oh1 * (rank1 + off_slot), axis=0, keepdims=True)
    pos_ref[0:1, :] = pos0.astype(jnp.int32)
    pos_ref[1:2, :] = pos1.astype(jnp.int32)

    nact = jnp.sum(nblk_i, axis=(0, 1), keepdims=False).reshape(1, 1)
    nsteps = nact * NJ
    nst_ref[...] = nsteps

    s_i = lax.broadcasted_iota(jnp.int32, (1, NSTEP), 1)
    sf = jnp.minimum(s_i, nsteps - 1).astype(jnp.float32)
    steps_inc = cum_incl * float(NJ)
    steps_exc = off_blk * float(NJ)
    e_of_s = jnp.sum((steps_inc <= sf).astype(jnp.float32),
                     axis=0, keepdims=True)
    e16 = lax.broadcasted_iota(jnp.int32, (NE, NSTEP), 0)
    oh_es = (e16 == e_of_s.astype(jnp.int32)).astype(jnp.float32)
    nblk_s = jnp.sum(oh_es * nblk, axis=0, keepdims=True)
    sexc_s = jnp.sum(oh_es * steps_exc, axis=0, keepdims=True)
    boff_s = jnp.sum(oh_es * off_blk, axis=0, keepdims=True)
    local = sf - sexc_s
    j_s = jnp.zeros((1, NSTEP), jnp.float32)
    for k in range(1, NJ):
        j_s = j_s + (local >= float(k) * nblk_s).astype(jnp.float32)
    b_s = local - j_s * nblk_s
    tbl_ref[0:1, :] = (boff_s + b_s).astype(jnp.int32)
    tbl_ref[1:2, :] = j_s.astype(jnp.int32)
    tbl_ref[2:3, :] = e_of_s.astype(jnp.int32)
    tbl_ref[3:4, :] = b_s.astype(jnp.int32)
    tbl_ref[4:5, :] = (j_s == 0.0).astype(jnp.int32)
    tbl_ref[5:6, :] = (j_s == float(NJ - 1)).astype(jnp.int32)
    tbl_ref[6:7, :] = jnp.zeros((1, NSTEP), jnp.int32)
    tbl_ref[7:8, :] = jnp.zeros((1, NSTEP), jnp.int32)


def _prep(sel0_row, sel1_row):
    return pl.pallas_call(
        _prep_body,
        out_shape=(
            jax.ShapeDtypeStruct((NK, NT), jnp.int32),
            jax.ShapeDtypeStruct((8, NSTEP), jnp.int32),
            jax.ShapeDtypeStruct((1, 1), jnp.int32),
        ),
    )(sel0_row, sel1_row)


_TPW = NT // 32


def _sc_dispatch(x, pos):
    @functools.partial(
        pl.kernel,
        out_type=jax.ShapeDtypeStruct((MPAD, DM), jnp.float32),
        mesh=plsc.VectorSubcoreMesh(core_axis_name="c", subcore_axis_name="s"),
        scratch_types=[
            pltpu.VMEM((_TPW, DM), jnp.float32),
            pltpu.VMEM((_TPW,), jnp.int32),
            pltpu.SemaphoreType.DMA,
        ],
    )
    def body(x_hbm, pos_hbm, xs_hbm, xtile, idx, sem):
        wid = lax.axis_index("s") * 2 + lax.axis_index("c")
        base = wid * _TPW
        pltpu.sync_copy(x_hbm.at[pl.ds(base, _TPW)], xtile)
        pltpu.sync_copy(pos_hbm.at[0, pl.ds(base, _TPW)], idx)
        pltpu.async_copy(xtile, xs_hbm.at[idx], sem).wait()
        pltpu.sync_copy(pos_hbm.at[1, pl.ds(base, _TPW)], idx)
        pltpu.async_copy(xtile, xs_hbm.at[idx], sem).wait()

    return body(x, pos)


def _sc_gather(ys, pos):
    @functools.partial(
        pl.kernel,
        out_type=(
            jax.ShapeDtypeStruct((NT, DM), jnp.float32),
            jax.ShapeDtypeStruct((NT, DM), jnp.float32),
        ),
        mesh=plsc.VectorSubcoreMesh(core_axis_name="c", subcore_axis_name="s"),
        scratch_types=[
            pltpu.VMEM((_TPW, DM), jnp.float32),
            pltpu.VMEM((_TPW,), jnp.int32),
            pltpu.SemaphoreType.DMA,
        ],
    )
    def body(ys_hbm, pos_hbm, y0_hbm, y1_hbm, buf, idx, sem):
        wid = lax.axis_index("s") * 2 + lax.axis_index("c")
        base = wid * _TPW
        pltpu.sync_copy(pos_hbm.at[0, pl.ds(base, _TPW)], idx)
        pltpu.async_copy(ys_hbm.at[idx], buf, sem).wait()
        pltpu.sync_copy(buf, y0_hbm.at[pl.ds(base, _TPW)])
        pltpu.sync_copy(pos_hbm.at[1, pl.ds(base, _TPW)], idx)
        pltpu.async_copy(ys_hbm.at[idx], buf, sem).wait()
        pltpu.sync_copy(buf, y1_hbm.at[pl.ds(base, _TPW)])

    return body(ys, pos)


def _mlp_body(tbl_ref, nst_ref, xs_ref, wfc_ref, wproj_ref, out_ref, acc_ref):
    s = pl.program_id(0)

    @pl.when(s < nst_ref[0])
    def _():
        x = xs_ref[...]
        h = lax.dot_general(x, wfc_ref[0], (((1,), (1,)), ((), ())),
                            preferred_element_type=jnp.float32)
        h = jax.nn.gelu(h)
        y = lax.dot_general(h, wproj_ref[0], (((1,), (1,)), ((), ())),
                            preferred_element_type=jnp.float32)
        off = tbl_ref[3, s] * BM

        @pl.when(tbl_ref[4, s] == 1)
        def _():
            acc_ref[pl.ds(off, BM), :] = y

        @pl.when(tbl_ref[4, s] == 0)
        def _():
            acc_ref[pl.ds(off, BM), :] += y

        @pl.when(tbl_ref[5, s] == 1)
        def _():
            out_ref[...] = acc_ref[pl.ds(off, BM), :]


def _grouped_mlp(xs, wfc, wproj, tbl, nst):
    grid_spec = pltpu.PrefetchScalarGridSpec(
        num_scalar_prefetch=2,
        grid=(NSTEP,),
        in_specs=[
            pl.BlockSpec((BM, DM), lambda s, tbl, nst: (tbl[0, s], 0)),
            pl.BlockSpec((1, DFN, DM),
                         lambda s, tbl, nst: (tbl[2, s], tbl[1, s], 0)),
            pl.BlockSpec((1, DM, DFN),
                         lambda s, tbl, nst: (tbl[2, s], 0, tbl[1, s])),
        ],
        out_specs=pl.BlockSpec(
            (BM, DM),
            lambda s, tbl, nst: (jnp.where(tbl[5, s] == 1, tbl[0, s], NBLK), 0)),
        scratch_shapes=[pltpu.VMEM((NE * BM, DM), jnp.float32)],
    )
    full = pl.pallas_call(
        _mlp_body,
        grid_spec=grid_spec,
        out_shape=jax.ShapeDtypeStruct((MPAD + BM, DM), jnp.float32),
    )(tbl, nst, xs, wfc, wproj)
    return full


def _combine_body(y0_ref, y1_ref, w0_ref, w1_ref, out_ref):
    out_ref[...] = w0_ref[...] * y0_ref[...] + w1_ref[...] * y1_ref[...]


def _combine(y0, y1, w0, w1):
    nb = 8
    bt = NT // nb
    return pl.pallas_call(
        _combine_body,
        grid=(nb,),
        in_specs=[
            pl.BlockSpec((bt, DM), lambda i: (i, 0)),
            pl.BlockSpec((bt, DM), lambda i: (i, 0)),
            pl.BlockSpec((bt, 1), lambda i: (i, 0)),
            pl.BlockSpec((bt, 1), lambda i: (i, 0)),
        ],
        out_specs=pl.BlockSpec((bt, DM), lambda i: (i, 0)),
        out_shape=jax.ShapeDtypeStruct((NT, DM), jnp.float32),
    )(y0, y1, w0, w1)


def kernel(hidden_states, Wg, Wfc, Wproj):
    Bc, Sc, Dc = hidden_states.shape
    x = hidden_states.reshape(-1, Dc)

    router_logits, sel0, sel1, w0, w1 = _router(x, Wg)

    pos, tbl, nst = _prep(sel0.reshape(1, NT), sel1.reshape(1, NT))

    xs = _sc_dispatch(x, pos)
    ys = _grouped_mlp(xs, Wfc, Wproj, tbl, nst.reshape(1))
    y0, y1 = _sc_gather(ys, pos)
    out = _combine(y0, y1, w0, w1)

    return out.reshape(Bc, Sc, Dc), router_logits


# speedup vs baseline: 1.1193x; 1.0008x over previous
import functools

import jax
import jax.numpy as jnp
from jax import lax
from jax.experimental import pallas as pl
from jax.experimental.pallas import tpu as pltpu
from jax.experimental.pallas import tpu_sc as plsc

NE = 16
NK = 2
DM = 1024
DF = 4096
NT = 2048
NP = NT * NK
BM = 256
NBLK = 32
MPAD = NBLK * BM
DFN = 2048
NJ = DF // DFN
NSTEP = NBLK * NJ

_HI = jax.lax.Precision.HIGHEST


def _router_body(x_ref, wg_ref, logits_ref, sel0_ref, sel1_ref, w0_ref, w1_ref):
    x = x_ref[...]
    logits = lax.dot_general(x, wg_ref[...], (((1,), (1,)), ((), ())),
                             preferred_element_type=jnp.float32)
    logits_ref[...] = logits
    lane = lax.broadcasted_iota(jnp.int32, (NT, NE), 1)
    neg = jnp.float32(-1e30)
    l = logits
    big = jnp.int32(1 << 30)
    m1 = jnp.max(l, axis=1, keepdims=True)
    a1 = jnp.min(jnp.where(l == m1, lane, big), axis=1, keepdims=True)
    l2 = jnp.where(lane == a1, neg, l)
    m2 = jnp.max(l2, axis=1, keepdims=True)
    a2 = jnp.min(jnp.where(l2 == m2, lane, big), axis=1, keepdims=True)
    w0 = 1.0 / (1.0 + jnp.exp(m2 - m1))
    sel0_ref[...] = a1
    sel1_ref[...] = a2
    w0_ref[...] = w0
    w1_ref[...] = 1.0 - w0


def _router(x, wg_pad):
    return pl.pallas_call(
        _router_body,
        out_shape=(
            jax.ShapeDtypeStruct((NT, NE), jnp.float32),
            jax.ShapeDtypeStruct((NT, 1), jnp.int32),
            jax.ShapeDtypeStruct((NT, 1), jnp.int32),
            jax.ShapeDtypeStruct((NT, 1), jnp.float32),
            jax.ShapeDtypeStruct((NT, 1), jnp.float32),
        ),
    )(x, wg_pad)


def _prep_body(sel0_ref, sel1_ref, pos_ref, tbl_ref, nst_ref):
    e_iota = lax.broadcasted_iota(jnp.int32, (NE, NT), 0)
    oh0 = (sel0_ref[...] == e_iota).astype(jnp.float32)
    oh1 = (sel1_ref[...] == e_iota).astype(jnp.float32)

    C = 512
    r = lax.broadcasted_iota(jnp.int32, (C, C), 0)
    cc = lax.broadcasted_iota(jnp.int32, (C, C), 1)
    u = (r < cc).astype(jnp.float32)

    def cum_excl(oh):
        parts = []
        carry = jnp.zeros((NE, 1), jnp.float32)
        for i in range(NT // C):
            blk = oh[:, i * C:(i + 1) * C]
            parts.append(lax.dot_general(blk, u, (((1,), (0,)), ((), ())),
                                         preferred_element_type=jnp.float32,
                                         precision=_HI) + carry)
            carry = carry + jnp.sum(blk, axis=1, keepdims=True)
        return jnp.concatenate(parts, axis=1), carry

    rank0, cnt0 = cum_excl(oh0)
    rank1, cnt1 = cum_excl(oh1)
    rank1 = rank1 + cnt0
    cnt = cnt0 + cnt1

    cnt_i = cnt.astype(jnp.int32)
    nblk_i = (cnt_i + (BM - 1)) >> 8
    nblk = nblk_i.astype(jnp.float32)

    tri = lax.broadcasted_iota(jnp.int32, (NE, NE), 1)
    row = lax.broadcasted_iota(jnp.int32, (NE, NE), 0)
    l_strict = (tri < row).astype(jnp.float32)
    l_incl = (tri <= row).astype(jnp.float32)
    off_blk = lax.dot_general(l_strict, nblk, (((1,), (0,)), ((), ())),
                              preferred_element_type=jnp.float32, precision=_HI)
    cum_incl = lax.dot_general(l_incl, nblk, (((1,), (0,)), ((), ())),
                               preferred_element_type=jnp.float32, precision=_HI)
    off_slot = off_blk * float(BM)

    pos0 = jnp.sum(oh0 * (rank0 + off_slot), axis=0, keepdims=True)
    pos1 = jnp.sum(oh1 * (rank1 + off_slot), axis=0, keepdims=True)
    pos_ref[0:1, :] = pos0.astype(jnp.int32)
    pos_ref[1:2, :] = pos1.astype(jnp.int32)

    nact = jnp.sum(nblk_i, axis=(0, 1), keepdims=False).reshape(1, 1)
    nsteps = nact * NJ
    nst_ref[...] = nsteps

    s_i = lax.broadcasted_iota(jnp.int32, (1, NSTEP), 1)
    sf = jnp.minimum(s_i, nsteps - 1).astype(jnp.float32)
    steps_inc = cum_incl * float(NJ)
    steps_exc = off_blk * float(NJ)
    e_of_s = jnp.sum((steps_inc <= sf).astype(jnp.float32),
                     axis=0, keepdims=True)
    e16 = lax.broadcasted_iota(jnp.int32, (NE, NSTEP), 0)
    oh_es = (e16 == e_of_s.astype(jnp.int32)).astype(jnp.float32)
    nblk_s = jnp.sum(oh_es * nblk, axis=0, keepdims=True)
    sexc_s = jnp.sum(oh_es * steps_exc, axis=0, keepdims=True)
    boff_s = jnp.sum(oh_es * off_blk, axis=0, keepdims=True)
    local = sf - sexc_s
    j_s = jnp.zeros((1, NSTEP), jnp.float32)
    for k in range(1, NJ):
        j_s = j_s + (local >= float(k) * nblk_s).astype(jnp.float32)
    b_s = local - j_s * nblk_s
    tbl_ref[0:1, :] = (boff_s + b_s).astype(jnp.int32)
    tbl_ref[1:2, :] = j_s.astype(jnp.int32)
    tbl_ref[2:3, :] = e_of_s.astype(jnp.int32)
    tbl_ref[3:4, :] = b_s.astype(jnp.int32)
    tbl_ref[4:5, :] = (j_s == 0.0).astype(jnp.int32)
    tbl_ref[5:6, :] = (j_s == float(NJ - 1)).astype(jnp.int32)
    tbl_ref[6:7, :] = jnp.zeros((1, NSTEP), jnp.int32)
    tbl_ref[7:8, :] = jnp.zeros((1, NSTEP), jnp.int32)


def _prep(sel0_row, sel1_row):
    return pl.pallas_call(
        _prep_body,
        out_shape=(
            jax.ShapeDtypeStruct((NK, NT), jnp.int32),
            jax.ShapeDtypeStruct((8, NSTEP), jnp.int32),
            jax.ShapeDtypeStruct((1, 1), jnp.int32),
        ),
    )(sel0_row, sel1_row)


_TPW = NT // 32


def _sc_dispatch(x, pos):
    @functools.partial(
        pl.kernel,
        out_type=jax.ShapeDtypeStruct((MPAD, DM), jnp.float32),
        mesh=plsc.VectorSubcoreMesh(core_axis_name="c", subcore_axis_name="s"),
        scratch_types=[
            pltpu.VMEM((_TPW, DM), jnp.float32),
            pltpu.VMEM((_TPW,), jnp.int32),
            pltpu.SemaphoreType.DMA,
        ],
    )
    def body(x_hbm, pos_hbm, xs_hbm, xtile, idx, sem):
        wid = lax.axis_index("s") * 2 + lax.axis_index("c")
        base = wid * _TPW
        pltpu.sync_copy(x_hbm.at[pl.ds(base, _TPW)], xtile)
        pltpu.sync_copy(pos_hbm.at[0, pl.ds(base, _TPW)], idx)
        pltpu.async_copy(xtile, xs_hbm.at[idx], sem).wait()
        pltpu.sync_copy(pos_hbm.at[1, pl.ds(base, _TPW)], idx)
        pltpu.async_copy(xtile, xs_hbm.at[idx], sem).wait()

    return body(x, pos)


def _sc_gather(ys, pos):
    @functools.partial(
        pl.kernel,
        out_type=(
            jax.ShapeDtypeStruct((NT, DM), jnp.float32),
            jax.ShapeDtypeStruct((NT, DM), jnp.float32),
        ),
        mesh=plsc.VectorSubcoreMesh(core_axis_name="c", subcore_axis_name="s"),
        scratch_types=[
            pltpu.VMEM((_TPW, DM), jnp.float32),
            pltpu.VMEM((_TPW,), jnp.int32),
            pltpu.SemaphoreType.DMA,
        ],
    )
    def body(ys_hbm, pos_hbm, y0_hbm, y1_hbm, buf, idx, sem):
        wid = lax.axis_index("s") * 2 + lax.axis_index("c")
        base = wid * _TPW
        pltpu.sync_copy(pos_hbm.at[0, pl.ds(base, _TPW)], idx)
        pltpu.async_copy(ys_hbm.at[idx], buf, sem).wait()
        pltpu.sync_copy(buf, y0_hbm.at[pl.ds(base, _TPW)])
        pltpu.sync_copy(pos_hbm.at[1, pl.ds(base, _TPW)], idx)
        pltpu.async_copy(ys_hbm.at[idx], buf, sem).wait()
        pltpu.sync_copy(buf, y1_hbm.at[pl.ds(base, _TPW)])

    return body(ys, pos)


def _mlp_body(tbl_ref, nst_ref, xs_ref, wfc_ref, wproj_ref, out_ref, acc_ref):
    s = pl.program_id(0)

    @pl.when(s < nst_ref[0])
    def _():
        x = xs_ref[...]
        h = lax.dot_general(x, wfc_ref[0], (((1,), (1,)), ((), ())),
                            preferred_element_type=jnp.float32)
        h = jax.nn.gelu(h)
        y = lax.dot_general(h, wproj_ref[0], (((1,), (1,)), ((), ())),
                            preferred_element_type=jnp.float32)
        off = tbl_ref[3, s] * BM

        @pl.when(tbl_ref[4, s] == 1)
        def _():
            acc_ref[pl.ds(off, BM), :] = y

        @pl.when(tbl_ref[4, s] == 0)
        def _():
            acc_ref[pl.ds(off, BM), :] += y

        @pl.when(tbl_ref[5, s] == 1)
        def _():
            out_ref[...] = acc_ref[pl.ds(off, BM), :]


def _grouped_mlp(xs, wfc, wproj, tbl, nst):
    grid_spec = pltpu.PrefetchScalarGridSpec(
        num_scalar_prefetch=2,
        grid=(NSTEP,),
        in_specs=[
            pl.BlockSpec((BM, DM), lambda s, tbl, nst: (tbl[0, s], 0)),
            pl.BlockSpec((1, DFN, DM),
                         lambda s, tbl, nst: (tbl[2, s], tbl[1, s], 0)),
            pl.BlockSpec((1, DM, DFN),
                         lambda s, tbl, nst: (tbl[2, s], 0, tbl[1, s])),
        ],
        out_specs=pl.BlockSpec(
            (BM, DM),
            lambda s, tbl, nst: (jnp.where(tbl[5, s] == 1, tbl[0, s], NBLK), 0)),
        scratch_shapes=[pltpu.VMEM((NE * BM, DM), jnp.float32)],
    )
    full = pl.pallas_call(
        _mlp_body,
        grid_spec=grid_spec,
        out_shape=jax.ShapeDtypeStruct((MPAD + BM, DM), jnp.float32),
    )(tbl, nst, xs, wfc, wproj)
    return full


def _combine_body(y0_ref, y1_ref, w0_ref, w1_ref, out_ref):
    out_ref[...] = w0_ref[...] * y0_ref[...] + w1_ref[...] * y1_ref[...]


def _combine(y0, y1, w0, w1):
    nb = 8
    bt = NT // nb
    return pl.pallas_call(
        _combine_body,
        grid=(nb,),
        in_specs=[
            pl.BlockSpec((bt, DM), lambda i: (i, 0)),
            pl.BlockSpec((bt, DM), lambda i: (i, 0)),
            pl.BlockSpec((bt, 1), lambda i: (i, 0)),
            pl.BlockSpec((bt, 1), lambda i: (i, 0)),
        ],
        out_specs=pl.BlockSpec((bt, DM), lambda i: (i, 0)),
        out_shape=jax.ShapeDtypeStruct((NT, DM), jnp.float32),
    )(y0, y1, w0, w1)


def kernel(hidden_states, Wg, Wfc, Wproj):
    Bc, Sc, Dc = hidden_states.shape
    x = hidden_states.reshape(-1, Dc)

    router_logits, sel0, sel1, w0, w1 = _router(x, Wg)

    pos, tbl, nst = _prep(sel0.reshape(1, NT), sel1.reshape(1, NT))

    xs = _sc_dispatch(x, pos)
    ys = _grouped_mlp(xs, Wfc, Wproj, tbl, nst.reshape(1))
    y0, y1 = _sc_gather(ys, pos)
    out = _combine(y0, y1, w0, w1)

    return out.reshape(Bc, Sc, Dc), router_logits
